# baseline (speedup 1.0000x reference)
import functools

import jax
import jax.numpy as jnp
from jax import lax
from jax.experimental import pallas as pl
from jax.experimental.pallas import tpu as pltpu
from jax.experimental.pallas import tpu_sc as plsc

L = 16
UNROLL = 2


def _unpack_xy(w):
    x = plsc.bitcast(lax.shift_left(w, 16), jnp.float32)
    y = plsc.bitcast(w, jnp.float32)
    return x, y


def _make_sc_kernel(n_nodes, p, nw, chunk):
    per_w = p // nw
    nchunk = per_w // chunk
    mesh = plsc.VectorSubcoreMesh(core_axis_name="c", subcore_axis_name="s")

    @functools.partial(
        pl.kernel,
        out_type=jax.ShapeDtypeStruct((nw, L), jnp.float32),
        mesh=mesh,
        compiler_params=pltpu.CompilerParams(needs_layout_passes=False),
        scratch_types=[
            pltpu.VMEM((n_nodes,), jnp.int32),
            pltpu.VMEM((chunk,), jnp.int32),
            pltpu.VMEM((chunk,), jnp.int32),
            pltpu.VMEM((chunk,), jnp.int32),
            pltpu.VMEM((chunk,), jnp.int32),
            pltpu.VMEM((chunk,), jnp.int32),
            pltpu.VMEM((chunk,), jnp.int32),
            pltpu.VMEM((L,), jnp.float32),
            pltpu.SemaphoreType.DMA,
            pltpu.SemaphoreType.DMA,
            pltpu.SemaphoreType.DMA,
        ],
    )
    def sc_kernel(tbl_hbm, vi0_hbm, vi1_hbm, vi2_hbm, out_hbm,
                  tbl_v, a0, a1, a2, b0, b1, b2, acc_v, sem_a, sem_b, sem_t):
        wid = lax.axis_index("s") * 2 + lax.axis_index("c")
        base = wid * per_w
        tbl_cp = pltpu.async_copy(tbl_hbm, tbl_v, sem_t)

        def start3(off, d0, d1, d2, sem):
            pltpu.async_copy(vi0_hbm.at[pl.ds(off, chunk)], d0, sem)
            pltpu.async_copy(vi1_hbm.at[pl.ds(off, chunk)], d1, sem)
            pltpu.async_copy(vi2_hbm.at[pl.ds(off, chunk)], d2, sem)

        def wait3(d0, d1, d2, sem):
            pltpu.make_async_copy(vi0_hbm.at[pl.ds(0, chunk)], d0, sem).wait()
            pltpu.make_async_copy(vi1_hbm.at[pl.ds(0, chunk)], d1, sem).wait()
            pltpu.make_async_copy(vi2_hbm.at[pl.ds(0, chunk)], d2, sem).wait()

        def compute(idx0_v, idx1_v, idx2_v, acc):
            def one(s):
                i0 = idx0_v[pl.ds(s, L)]
                i1 = idx1_v[pl.ds(s, L)]
                i2 = idx2_v[pl.ds(s, L)]
                w0 = plsc.bitcast(plsc.load_gather(tbl_v, [i0]), jnp.bfloat16)
                w1 = plsc.bitcast(plsc.load_gather(tbl_v, [i1]), jnp.bfloat16)
                w2 = plsc.bitcast(plsc.load_gather(tbl_v, [i2]), jnp.bfloat16)
                ax, ay = _unpack_xy(plsc.bitcast(w1 - w0, jnp.int32))
                bx, by = _unpack_xy(plsc.bitcast(w2 - w0, jnp.int32))
                na = ax * ax + ay * ay
                nb = bx * bx + by * by
                dot = ax * bx + ay * by
                m = jnp.maximum(na * nb, jnp.float32(1e-30))
                r = plsc.bitcast(
                    jnp.int32(0x5F3759DF) - lax.shift_right_arithmetic(
                        plsc.bitcast(m, jnp.int32), 1),
                    jnp.float32)
                r = r * (jnp.float32(1.5008) - jnp.float32(0.5) * m * r * r)
                return dot * r

            def inner(s, acc):
                return acc + one(s)

            return plsc.parallel_loop(0, chunk, L, unroll=UNROLL, carry=acc)(inner)

        start3(base, a0, a1, a2, sem_a)
        tbl_cp.wait()

        def pair_body(h, acc):
            c = h * 2
            start3(base + (c + 1) * chunk, b0, b1, b2, sem_b)
            wait3(a0, a1, a2, sem_a)
            acc = compute(a0, a1, a2, acc)
            nxt = jnp.minimum(c + 2, nchunk - 1)
            start3(base + nxt * chunk, a0, a1, a2, sem_a)
            wait3(b0, b1, b2, sem_b)
            acc = compute(b0, b1, b2, acc)
            return acc

        acc = lax.fori_loop(0, nchunk // 2, pair_body,
                            jnp.zeros((L,), jnp.float32))
        wait3(a0, a1, a2, sem_a)
        acc_v[...] = acc
        pltpu.sync_copy(acc_v, out_hbm.at[wid])

    return sc_kernel


def kernel(node_pos, vi0, vi1, vi2):
    n_nodes = node_pos.shape[0]
    p = vi0.shape[0]
    u = lax.bitcast_convert_type(node_pos.astype(jnp.bfloat16), jnp.uint16)
    packed = (u[:, 0].astype(jnp.uint32)
              | (u[:, 1].astype(jnp.uint32) << 16)).astype(jnp.int32)
    sc = _make_sc_kernel(n_nodes, p, nw=32, chunk=4096)
    partial = sc(packed, vi0, vi1, vi2)
    return jnp.sum(partial)


# speedup vs baseline: 1.0251x; 1.0251x over previous
import functools

import jax
import jax.numpy as jnp
from jax import lax
from jax.experimental import pallas as pl
from jax.experimental.pallas import tpu as pltpu
from jax.experimental.pallas import tpu_sc as plsc

L = 16
UNROLL = 4


def _unpack_xy(w):
    x = plsc.bitcast(lax.shift_left(w, 16), jnp.float32)
    y = plsc.bitcast(w, jnp.float32)
    return x, y


def _make_sc_kernel(n_nodes, p, nw, chunk):
    per_w = p // nw
    nchunk = per_w // chunk
    mesh = plsc.VectorSubcoreMesh(core_axis_name="c", subcore_axis_name="s")

    @functools.partial(
        pl.kernel,
        out_type=jax.ShapeDtypeStruct((nw, L), jnp.float32),
        mesh=mesh,
        compiler_params=pltpu.CompilerParams(needs_layout_passes=False),
        scratch_types=[
            pltpu.VMEM((n_nodes,), jnp.int32),
            pltpu.VMEM((chunk,), jnp.int32),
            pltpu.VMEM((chunk,), jnp.int32),
            pltpu.VMEM((chunk,), jnp.int32),
            pltpu.VMEM((chunk,), jnp.int32),
            pltpu.VMEM((chunk,), jnp.int32),
            pltpu.VMEM((chunk,), jnp.int32),
            pltpu.VMEM((L,), jnp.float32),
            pltpu.SemaphoreType.DMA,
            pltpu.SemaphoreType.DMA,
            pltpu.SemaphoreType.DMA,
        ],
    )
    def sc_kernel(tbl_hbm, vi0_hbm, vi1_hbm, vi2_hbm, out_hbm,
                  tbl_v, a0, a1, a2, b0, b1, b2, acc_v, sem_a, sem_b, sem_t):
        wid = lax.axis_index("s") * 2 + lax.axis_index("c")
        base = wid * per_w
        tbl_cp = pltpu.async_copy(tbl_hbm, tbl_v, sem_t)

        def start3(off, d0, d1, d2, sem):
            pltpu.async_copy(vi0_hbm.at[pl.ds(off, chunk)], d0, sem)
            pltpu.async_copy(vi1_hbm.at[pl.ds(off, chunk)], d1, sem)
            pltpu.async_copy(vi2_hbm.at[pl.ds(off, chunk)], d2, sem)

        def wait3(d0, d1, d2, sem):
            pltpu.make_async_copy(vi0_hbm.at[pl.ds(0, chunk)], d0, sem).wait()
            pltpu.make_async_copy(vi1_hbm.at[pl.ds(0, chunk)], d1, sem).wait()
            pltpu.make_async_copy(vi2_hbm.at[pl.ds(0, chunk)], d2, sem).wait()

        def compute(idx0_v, idx1_v, idx2_v, acc):
            def one(s):
                i0 = idx0_v[pl.ds(s, L)]
                i1 = idx1_v[pl.ds(s, L)]
                i2 = idx2_v[pl.ds(s, L)]
                w0 = plsc.bitcast(plsc.load_gather(tbl_v, [i0]), jnp.bfloat16)
                w1 = plsc.bitcast(plsc.load_gather(tbl_v, [i1]), jnp.bfloat16)
                w2 = plsc.bitcast(plsc.load_gather(tbl_v, [i2]), jnp.bfloat16)
                ax, ay = _unpack_xy(plsc.bitcast(w1 - w0, jnp.int32))
                bx, by = _unpack_xy(plsc.bitcast(w2 - w0, jnp.int32))
                dot = ax * bx + ay * by
                crs = ax * by - ay * bx
                m = jnp.maximum(dot * dot + crs * crs, jnp.float32(1e-30))
                r = plsc.bitcast(
                    jnp.int32(0x5F3759DF) - lax.shift_right_arithmetic(
                        plsc.bitcast(m, jnp.int32), 1),
                    jnp.float32)
                u = m * (r * r)
                r = r * (jnp.float32(1.5008) - jnp.float32(0.5) * u)
                return dot * r

            def inner(s, acc):
                return acc + one(s)

            return plsc.parallel_loop(0, chunk, L, unroll=UNROLL, carry=acc)(inner)

        start3(base, a0, a1, a2, sem_a)
        tbl_cp.wait()

        def pair_body(h, acc):
            c = h * 2
            start3(base + (c + 1) * chunk, b0, b1, b2, sem_b)
            wait3(a0, a1, a2, sem_a)
            acc = compute(a0, a1, a2, acc)
            nxt = jnp.minimum(c + 2, nchunk - 1)
            start3(base + nxt * chunk, a0, a1, a2, sem_a)
            wait3(b0, b1, b2, sem_b)
            acc = compute(b0, b1, b2, acc)
            return acc

        acc = lax.fori_loop(0, nchunk // 2, pair_body,
                            jnp.zeros((L,), jnp.float32))
        wait3(a0, a1, a2, sem_a)
        acc_v[...] = acc
        pltpu.sync_copy(acc_v, out_hbm.at[wid])

    return sc_kernel


def kernel(node_pos, vi0, vi1, vi2):
    n_nodes = node_pos.shape[0]
    p = vi0.shape[0]
    u = lax.bitcast_convert_type(node_pos.astype(jnp.bfloat16), jnp.uint16)
    packed = (u[:, 0].astype(jnp.uint32)
              | (u[:, 1].astype(jnp.uint32) << 16)).astype(jnp.int32)
    sc = _make_sc_kernel(n_nodes, p, nw=32, chunk=4096)
    partial = sc(packed, vi0, vi1, vi2)
    return jnp.sum(partial)


# speedup vs baseline: 1.0723x; 1.0461x over previous
import functools

import jax
import jax.numpy as jnp
from jax import lax
from jax.experimental import pallas as pl
from jax.experimental.pallas import tpu as pltpu
from jax.experimental.pallas import tpu_sc as plsc

L = 16
UNROLL = 4


def _unpack_xy(w):
    x = plsc.bitcast(lax.shift_left(w, 16), jnp.float32)
    y = plsc.bitcast(w, jnp.float32)
    return x, y


def _make_sc_kernel(n_nodes, p, nw, chunk):
    per_w = p // nw
    nchunk = per_w // chunk
    mesh = plsc.VectorSubcoreMesh(core_axis_name="c", subcore_axis_name="s")

    @functools.partial(
        pl.kernel,
        out_type=jax.ShapeDtypeStruct((nw, L), jnp.float32),
        mesh=mesh,
        compiler_params=pltpu.CompilerParams(needs_layout_passes=False),
        scratch_types=[
            pltpu.VMEM((n_nodes,), jnp.int32),
            pltpu.VMEM((chunk,), jnp.int32),
            pltpu.VMEM((chunk,), jnp.int32),
            pltpu.VMEM((chunk,), jnp.int32),
            pltpu.VMEM((chunk,), jnp.int32),
            pltpu.VMEM((chunk,), jnp.int32),
            pltpu.VMEM((chunk,), jnp.int32),
            pltpu.VMEM((L,), jnp.float32),
            pltpu.SemaphoreType.DMA,
            pltpu.SemaphoreType.DMA,
            pltpu.SemaphoreType.DMA,
        ],
    )
    def sc_kernel(tbl_hbm, vi0_hbm, vi1_hbm, vi2_hbm, out_hbm,
                  tbl_v, a0, a1, a2, b0, b1, b2, acc_v, sem_a, sem_b, sem_t):
        wid = lax.axis_index("s") * 2 + lax.axis_index("c")
        base = wid * per_w
        tbl_cp = pltpu.async_copy(tbl_hbm, tbl_v, sem_t)

        def start3(off, d0, d1, d2, sem):
            pltpu.async_copy(vi0_hbm.at[pl.ds(off, chunk)], d0, sem)
            pltpu.async_copy(vi1_hbm.at[pl.ds(off, chunk)], d1, sem)
            pltpu.async_copy(vi2_hbm.at[pl.ds(off, chunk)], d2, sem)

        def wait3(d0, d1, d2, sem):
            pltpu.make_async_copy(vi0_hbm.at[pl.ds(0, chunk)], d0, sem).wait()
            pltpu.make_async_copy(vi1_hbm.at[pl.ds(0, chunk)], d1, sem).wait()
            pltpu.make_async_copy(vi2_hbm.at[pl.ds(0, chunk)], d2, sem).wait()

        def compute(idx0_v, idx1_v, idx2_v, acc):
            def one(s):
                i0 = idx0_v[pl.ds(s, L)]
                i1 = idx1_v[pl.ds(s, L)]
                i2 = idx2_v[pl.ds(s, L)]
                w0 = plsc.bitcast(plsc.load_gather(tbl_v, [i0]), jnp.bfloat16)
                w1 = plsc.bitcast(plsc.load_gather(tbl_v, [i1]), jnp.bfloat16)
                w2 = plsc.bitcast(plsc.load_gather(tbl_v, [i2]), jnp.bfloat16)
                ax, ay = _unpack_xy(plsc.bitcast(w1 - w0, jnp.int32))
                bx, by = _unpack_xy(plsc.bitcast(w2 - w0, jnp.int32))
                dot = ax * bx + ay * by
                crs = ax * by - ay * bx
                m = dot * dot + crs * crs
                p = plsc.bitcast(
                    jnp.int32(0x5EF7A3B2) - lax.shift_right_arithmetic(
                        plsc.bitcast(m, jnp.int32), 1),
                    jnp.float32)
                v = p * (jnp.float32(1.501) - m * (p * p))
                return dot * v

            def inner(s, acc):
                return acc + one(s)

            return plsc.parallel_loop(0, chunk, L, unroll=UNROLL, carry=acc)(inner)

        start3(base, a0, a1, a2, sem_a)
        tbl_cp.wait()

        def pair_body(h, acc):
            c = h * 2
            start3(base + (c + 1) * chunk, b0, b1, b2, sem_b)
            wait3(a0, a1, a2, sem_a)
            acc = compute(a0, a1, a2, acc)
            nxt = jnp.minimum(c + 2, nchunk - 1)
            start3(base + nxt * chunk, a0, a1, a2, sem_a)
            wait3(b0, b1, b2, sem_b)
            acc = compute(b0, b1, b2, acc)
            return acc

        acc = lax.fori_loop(0, nchunk // 2, pair_body,
                            jnp.zeros((L,), jnp.float32))
        wait3(a0, a1, a2, sem_a)
        acc_v[...] = acc * jnp.float32(1.4142135623730951)
        pltpu.sync_copy(acc_v, out_hbm.at[wid])

    return sc_kernel


def kernel(node_pos, vi0, vi1, vi2):
    n_nodes = node_pos.shape[0]
    p = vi0.shape[0]
    u = lax.bitcast_convert_type(node_pos.astype(jnp.bfloat16), jnp.uint16)
    packed = (u[:, 0].astype(jnp.uint32)
              | (u[:, 1].astype(jnp.uint32) << 16)).astype(jnp.int32)
    sc = _make_sc_kernel(n_nodes, p, nw=32, chunk=4096)
    partial = sc(packed, vi0, vi1, vi2)
    return jnp.sum(partial)
